# Initial kernel scaffold; baseline (speedup 1.0000x reference)
import jax
import jax.numpy as jnp
from jax.experimental import pallas as pl


def kernel(node_feats, batch_idx, w_attn, b_attn, w_mask, b_mask):
    raise NotImplementedError("write your pallas kernel here")


# speedup vs baseline: 1.6031x; 1.6031x over previous
import jax
import jax.numpy as jnp
from jax import lax
from jax.experimental import pallas as pl
from jax.experimental.pallas import tpu as pltpu
from jax.experimental.pallas import tpu_sc as plsc

_N = 50000
_D = 256
_G = 256
_L = 16
_C = 80
_CHUNKS = _N // _C
_NW = 32
_K = -(-_CHUNKS // _NW)
_DK = _D // _L


def _sc_body(nf_hbm, bi_hbm, wa_hbm, wm_hbm, cb_hbm,
             partials_hbm, scores_hbm,
             rowbuf, idxbuf, scorebuf, accbuf, wabuf, wmbuf, cbbuf):
    wid = lax.axis_index("s") * 2 + lax.axis_index("c")
    pltpu.sync_copy(wa_hbm, wabuf)
    pltpu.sync_copy(wm_hbm, wmbuf)
    pltpu.sync_copy(cb_hbm, cbbuf)

    def _zero(i, carry):
        for k in range(_DK):
            accbuf[i, pl.ds(k * _L, _L)] = jnp.zeros((_L,), jnp.float32)
        return carry

    lax.fori_loop(0, _G, _zero, 0)

    cb_vec = cbbuf[pl.ds(0, _L)]
    b_a = cb_vec[0]
    b_m = cb_vec[1]
    lane_iota = lax.iota(jnp.int32, _L)

    start = wid * _K
    end = jnp.minimum(start + _K, _CHUNKS)

    def _chunk(j, carry):
        base = j * _C
        pltpu.sync_copy(nf_hbm.at[pl.ds(base, _C), :], rowbuf)
        pltpu.sync_copy(bi_hbm.at[pl.ds(base, _C)], idxbuf)

        def _group(g, gcarry):
            r0 = g * _L
            idx_vec = idxbuf[pl.ds(r0, _L)]
            sc_vec = jnp.zeros((_L,), jnp.float32)
            for r16 in range(_L):
                r = r0 + r16
                v = [rowbuf[r, pl.ds(k * _L, _L)] for k in range(_DK)]
                acc_a = v[0] * wabuf[pl.ds(0, _L)]
                acc_m = v[0] * wmbuf[pl.ds(0, _L)]
                for k in range(1, _DK):
                    acc_a = acc_a + v[k] * wabuf[pl.ds(k * _L, _L)]
                    acc_m = acc_m + v[k] * wmbuf[pl.ds(k * _L, _L)]
                sa = jnp.sum(acc_a) + b_a
                sm = jnp.sum(acc_m) + b_m
                sa_vec = jnp.full((_L,), sa, jnp.float32)
                sm_vec = jnp.full((_L,), sm, jnp.float32)
                score_vec = sa_vec / (1.0 + jnp.exp(-sm_vec))
                seg = idx_vec[r16]
                for k in range(_DK):
                    plsc.addupdate(accbuf.at[seg, pl.ds(k * _L, _L)],
                                   v[k] * score_vec)
                sc_vec = jnp.where(lane_iota == r16, score_vec, sc_vec)
            scorebuf[pl.ds(r0, _L)] = sc_vec
            return gcarry

        lax.fori_loop(0, _C // _L, _group, 0)
        pltpu.sync_copy(scorebuf, scores_hbm.at[pl.ds(base, _C)])
        return carry

    lax.fori_loop(start, end, _chunk, 0)
    pltpu.sync_copy(accbuf, partials_hbm.at[wid])


def _merge_body(p_ref, o_ref):
    o_ref[...] = jnp.sum(p_ref[...], axis=0)


@jax.jit
def kernel(node_feats, batch_idx, w_attn, b_attn, w_mask, b_mask):
    bi = batch_idx.astype(jnp.int32)
    wa = w_attn.reshape(_D).astype(jnp.float32)
    wm = w_mask.reshape(_D).astype(jnp.float32)
    cb = jnp.concatenate([b_attn.reshape(1).astype(jnp.float32),
                          b_mask.reshape(1).astype(jnp.float32),
                          jnp.zeros((_L - 2,), jnp.float32)])

    mesh = plsc.VectorSubcoreMesh(core_axis_name="c", subcore_axis_name="s")
    sc_call = pl.kernel(
        _sc_body,
        out_type=(jax.ShapeDtypeStruct((_NW, _G, _D), jnp.float32),
                  jax.ShapeDtypeStruct((_N,), jnp.float32)),
        mesh=mesh,
        compiler_params=pltpu.CompilerParams(needs_layout_passes=False),
        scratch_types=[
            pltpu.VMEM((_C, _D), jnp.float32),
            pltpu.VMEM((_C,), jnp.int32),
            pltpu.VMEM((_C,), jnp.float32),
            pltpu.VMEM((_G, _D), jnp.float32),
            pltpu.VMEM((_D,), jnp.float32),
            pltpu.VMEM((_D,), jnp.float32),
            pltpu.VMEM((_L,), jnp.float32),
        ],
    )
    partials, scores = sc_call(node_feats, bi, wa, wm, cb)

    pooled = pl.pallas_call(
        _merge_body,
        out_shape=jax.ShapeDtypeStruct((_G, _D), jnp.float32),
    )(partials)
    return pooled, scores


# speedup vs baseline: 3.3637x; 2.0982x over previous
import jax
import jax.numpy as jnp
from jax import lax
from jax.experimental import pallas as pl
from jax.experimental.pallas import tpu as pltpu
from jax.experimental.pallas import tpu_sc as plsc

_N = 50000
_D = 256
_G = 256
_L = 16
_C = 80
_CHUNKS = _N // _C
_NW = 32
_K = -(-_CHUNKS // _NW)
_DK = _D // _L
_TR = _K * _C
_PADN = _NW * _TR


def _sc_body(nf_hbm, bi_hbm, wa_hbm, wm_hbm, cb_hbm,
             partials_hbm, scores_hbm,
             rowbuf, idxbuf, scorebuf, accbuf, wabuf, wmbuf, cbbuf, sem_r):
    wid = lax.axis_index("s") * 2 + lax.axis_index("c")
    start = wid * _K
    end = jnp.minimum(start + _K, _CHUNKS)

    pltpu.sync_copy(wa_hbm, wabuf)
    pltpu.sync_copy(wm_hbm, wmbuf)
    pltpu.sync_copy(cb_hbm, cbbuf)
    pltpu.sync_copy(bi_hbm.at[pl.ds(start * _C, _TR)], idxbuf)

    def _zero(i, carry):
        for k in range(_DK):
            accbuf[i, pl.ds(k * _L, _L)] = jnp.zeros((_L,), jnp.float32)
        return carry

    lax.fori_loop(0, _G, _zero, 0)

    cb_vec = cbbuf[pl.ds(0, _L)]
    b_a = cb_vec[0]
    b_m = cb_vec[1]
    mask0 = lax.iota(jnp.int32, _L) == 0

    @pl.when(start < end)
    def _prologue():
        pltpu.make_async_copy(nf_hbm.at[pl.ds(start * _C, _C), :],
                              rowbuf.at[0], sem_r).start()

    def _chunk(j, carry):
        local = j - start
        buf = lax.rem(local, 2)
        pltpu.make_async_copy(nf_hbm.at[pl.ds(0, _C), :],
                              rowbuf.at[buf], sem_r).wait()

        @pl.when(j + 1 < end)
        def _next():
            pltpu.make_async_copy(nf_hbm.at[pl.ds((j + 1) * _C, _C), :],
                                  rowbuf.at[1 - buf], sem_r).start()

        base_local = local * _C

        @plsc.parallel_loop(0, _C // _L)
        def _group(g):
            r0 = g * _L
            idx_vec = idxbuf[pl.ds(base_local + r0, _L)]
            wa_vecs = [wabuf[pl.ds(k * _L, _L)] for k in range(_DK)]
            wm_vecs = [wmbuf[pl.ds(k * _L, _L)] for k in range(_DK)]
            for r16 in range(_L):
                r = r0 + r16
                v = [rowbuf[buf, r, pl.ds(k * _L, _L)] for k in range(_DK)]
                pa = [v[k] * wa_vecs[k] for k in range(4)]
                pm = [v[k] * wm_vecs[k] for k in range(4)]
                for k in range(4, _DK):
                    t = k & 3
                    pa[t] = pa[t] + v[k] * wa_vecs[k]
                    pm[t] = pm[t] + v[k] * wm_vecs[k]
                acc_a = (pa[0] + pa[1]) + (pa[2] + pa[3])
                acc_m = (pm[0] + pm[1]) + (pm[2] + pm[3])
                sa = jnp.sum(acc_a) + b_a
                sm = jnp.sum(acc_m) + b_m
                sa_vec = jnp.full((_L,), sa, jnp.float32)
                sm_vec = jnp.full((_L,), sm, jnp.float32)
                score_vec = sa_vec / (1.0 + jnp.exp(-sm_vec))
                seg = idx_vec[r16]
                for k in range(_DK):
                    plsc.addupdate(accbuf.at[seg, pl.ds(k * _L, _L)],
                                   v[k] * score_vec)
                plsc.store_scatter(
                    scorebuf,
                    [jnp.full((_L,), base_local + r, jnp.int32)],
                    score_vec, mask=mask0)

        return carry

    lax.fori_loop(start, end, _chunk, 0)
    pltpu.sync_copy(scorebuf, scores_hbm.at[pl.ds(start * _C, _TR)])
    pltpu.sync_copy(accbuf, partials_hbm.at[wid])


def _merge_body(p_ref, o_ref):
    o_ref[...] = jnp.sum(p_ref[...], axis=0)


@jax.jit
def kernel(node_feats, batch_idx, w_attn, b_attn, w_mask, b_mask):
    bi = batch_idx.astype(jnp.int32)
    bi_pad = jnp.concatenate(
        [bi, jnp.zeros((_PADN - _N,), jnp.int32)])
    wa = w_attn.reshape(_D).astype(jnp.float32)
    wm = w_mask.reshape(_D).astype(jnp.float32)
    cb = jnp.concatenate([b_attn.reshape(1).astype(jnp.float32),
                          b_mask.reshape(1).astype(jnp.float32),
                          jnp.zeros((_L - 2,), jnp.float32)])

    mesh = plsc.VectorSubcoreMesh(core_axis_name="c", subcore_axis_name="s")
    sc_call = pl.kernel(
        _sc_body,
        out_type=(jax.ShapeDtypeStruct((_NW, _G, _D), jnp.float32),
                  jax.ShapeDtypeStruct((_PADN,), jnp.float32)),
        mesh=mesh,
        compiler_params=pltpu.CompilerParams(needs_layout_passes=False),
        scratch_types=[
            pltpu.VMEM((2, _C, _D), jnp.float32),
            pltpu.VMEM((_TR,), jnp.int32),
            pltpu.VMEM((_TR,), jnp.float32),
            pltpu.VMEM((_G, _D), jnp.float32),
            pltpu.VMEM((_D,), jnp.float32),
            pltpu.VMEM((_D,), jnp.float32),
            pltpu.VMEM((_L,), jnp.float32),
            pltpu.SemaphoreType.DMA,
        ],
    )
    partials, scores_pad = sc_call(node_feats, bi_pad, wa, wm, cb)

    pooled = pl.pallas_call(
        _merge_body,
        out_shape=jax.ShapeDtypeStruct((_G, _D), jnp.float32),
    )(partials)
    return pooled, scores_pad[:_N]


# speedup vs baseline: 4.7269x; 1.4053x over previous
import jax
import jax.numpy as jnp
from jax import lax
from jax.experimental import pallas as pl
from jax.experimental.pallas import tpu as pltpu
from jax.experimental.pallas import tpu_sc as plsc

_N = 50000
_D = 256
_G = 256
_L = 16
_C = 80
_CHUNKS = _N // _C
_NW = 32
_K = -(-_CHUNKS // _NW)
_DK = _D // _L
_TR = _K * _C
_PADN = _NW * _TR


def _sc_body(nf_hbm, bi_hbm, wa_hbm, wm_hbm, cb_hbm,
             partials_hbm, scores_hbm,
             rowbuf, idxbuf, scorebuf, accbuf, wabuf, wmbuf, cbbuf, sem_r):
    wid = lax.axis_index("s") * 2 + lax.axis_index("c")
    start = wid * _K
    end = jnp.minimum(start + _K, _CHUNKS)

    pltpu.sync_copy(wa_hbm, wabuf)
    pltpu.sync_copy(wm_hbm, wmbuf)
    pltpu.sync_copy(cb_hbm, cbbuf)
    pltpu.sync_copy(bi_hbm.at[pl.ds(start * _C, _TR)], idxbuf.at[pl.ds(0, _TR)])

    def _zero(i, carry):
        for k in range(_DK):
            accbuf[i, pl.ds(k * _L, _L)] = jnp.zeros((_L,), jnp.float32)
        return carry

    lax.fori_loop(0, _G, _zero, 0)

    cb_vec = cbbuf[pl.ds(0, _L)]
    b_a = cb_vec[0]
    b_m = cb_vec[1]
    mask0 = lax.iota(jnp.int32, _L) == 0

    @pl.when(start < end)
    def _prologue():
        pltpu.make_async_copy(nf_hbm.at[pl.ds(start * _C, _C), :],
                              rowbuf.at[0], sem_r).start()

    def _chunk(j, carry):
        local = j - start
        buf = lax.rem(local, 2)
        pltpu.make_async_copy(nf_hbm.at[pl.ds(0, _C), :],
                              rowbuf.at[buf], sem_r).wait()

        @pl.when(j + 1 < end)
        def _next():
            pltpu.make_async_copy(nf_hbm.at[pl.ds((j + 1) * _C, _C), :],
                                  rowbuf.at[1 - buf], sem_r).start()

        base_local = local * _C

        @plsc.parallel_loop(0, _C // _L)
        def _group(g):
            r0 = g * _L
            wa_vecs = [wabuf[pl.ds(k * _L, _L)] for k in range(_DK)]
            wm_vecs = [wmbuf[pl.ds(k * _L, _L)] for k in range(_DK)]

            @plsc.parallel_loop(0, _L, unroll=_L)
            def _row(r16):
                r = r0 + r16
                v = [rowbuf[buf, r, pl.ds(k * _L, _L)] for k in range(_DK)]
                pa = [v[k] * wa_vecs[k] for k in range(4)]
                pm = [v[k] * wm_vecs[k] for k in range(4)]
                for k in range(4, _DK):
                    t = k & 3
                    pa[t] = pa[t] + v[k] * wa_vecs[k]
                    pm[t] = pm[t] + v[k] * wm_vecs[k]
                acc_a = (pa[0] + pa[1]) + (pa[2] + pa[3])
                acc_m = (pm[0] + pm[1]) + (pm[2] + pm[3])
                sa = jnp.sum(acc_a) + b_a
                sm = jnp.sum(acc_m) + b_m
                sa_vec = jnp.full((_L,), sa, jnp.float32)
                sm_vec = jnp.full((_L,), sm, jnp.float32)
                score_vec = sa_vec / (1.0 + jnp.exp(-sm_vec))
                seg = idxbuf[pl.ds(base_local + r, _L)][0]
                for k in range(_DK):
                    plsc.addupdate(accbuf.at[seg, pl.ds(k * _L, _L)],
                                   v[k] * score_vec)
                plsc.store_scatter(
                    scorebuf,
                    [jnp.full((_L,), base_local + r, jnp.int32)],
                    score_vec, mask=mask0)

        return carry

    lax.fori_loop(start, end, _chunk, 0)
    pltpu.sync_copy(scorebuf, scores_hbm.at[pl.ds(start * _C, _TR)])
    pltpu.sync_copy(accbuf, partials_hbm.at[wid])


def _merge_body(p_ref, o_ref):
    o_ref[...] = jnp.sum(p_ref[...], axis=0)


@jax.jit
def kernel(node_feats, batch_idx, w_attn, b_attn, w_mask, b_mask):
    bi = batch_idx.astype(jnp.int32)
    bi_pad = jnp.concatenate(
        [bi, jnp.zeros((_PADN - _N,), jnp.int32)])
    wa = w_attn.reshape(_D).astype(jnp.float32)
    wm = w_mask.reshape(_D).astype(jnp.float32)
    cb = jnp.concatenate([b_attn.reshape(1).astype(jnp.float32),
                          b_mask.reshape(1).astype(jnp.float32),
                          jnp.zeros((_L - 2,), jnp.float32)])

    mesh = plsc.VectorSubcoreMesh(core_axis_name="c", subcore_axis_name="s")
    sc_call = pl.kernel(
        _sc_body,
        out_type=(jax.ShapeDtypeStruct((_NW, _G, _D), jnp.float32),
                  jax.ShapeDtypeStruct((_PADN,), jnp.float32)),
        mesh=mesh,
        compiler_params=pltpu.CompilerParams(needs_layout_passes=False),
        scratch_types=[
            pltpu.VMEM((2, _C, _D), jnp.float32),
            pltpu.VMEM((_TR + _L,), jnp.int32),
            pltpu.VMEM((_TR,), jnp.float32),
            pltpu.VMEM((_G, _D), jnp.float32),
            pltpu.VMEM((_D,), jnp.float32),
            pltpu.VMEM((_D,), jnp.float32),
            pltpu.VMEM((_L,), jnp.float32),
            pltpu.SemaphoreType.DMA,
        ],
    )
    partials, scores_pad = sc_call(node_feats, bi_pad, wa, wm, cb)

    pooled = pl.pallas_call(
        _merge_body,
        out_shape=jax.ShapeDtypeStruct((_G, _D), jnp.float32),
    )(partials)
    return pooled, scores_pad[:_N]
